# Initial kernel scaffold; baseline (speedup 1.0000x reference)
import functools

import jax
import jax.numpy as jnp
from jax import lax
from jax.experimental import pallas as pl
from jax.experimental.pallas import tpu as pltpu
from jax.experimental.pallas import tpu_sc as plsc

B = 16384
F = 26
D = 16

_INFO = plsc.get_sparse_core_info()
NC = _INFO.num_cores
NS = _INFO.num_subcores
NW = NC * NS

CB = 64
ROWS = CB * F
G = ROWS // 128
B_PER_W = B // NW
NCHUNK = B_PER_W // CB


def _make_kernel():
    mesh = plsc.VectorSubcoreMesh(core_axis_name="c", subcore_axis_name="s")

    @functools.partial(
        pl.kernel,
        mesh=mesh,
        out_type=jax.ShapeDtypeStruct((B, D), jnp.float32),
        scratch_types=[
            pltpu.VMEM((G, 128), jnp.int32),
            pltpu.VMEM((CB, F), jnp.float32),
            pltpu.VMEM((ROWS, D), jnp.float32),
            pltpu.VMEM((CB, D), jnp.float32),
            pltpu.VMEM((D,), jnp.float32),
            pltpu.SemaphoreType.DMA,
        ],
    )
    def feature_linear(idx_hbm, val_hbm, table_hbm, bias_hbm, out_hbm,
                       idx_v, val_v, rows_v, out_v, bias_v, sem):
        wid = lax.axis_index("s") * NC + lax.axis_index("c")
        pltpu.sync_copy(bias_hbm, bias_v)

        for c in range(NCHUNK):
            base = wid * B_PER_W + c * CB
            goff = base * F // 128
            pltpu.sync_copy(idx_hbm.at[pl.ds(goff, G)], idx_v)
            pltpu.sync_copy(val_hbm.at[pl.ds(base, CB)], val_v)

            copies = []
            for g in range(G):
                copies.append(
                    pltpu.async_copy(
                        table_hbm.at[idx_v.at[g]],
                        rows_v.at[pl.ds(g * 128, 128)],
                        sem,
                    )
                )
            for cp in copies:
                cp.wait()

            def body(i, _):
                acc = bias_v[:]
                rb = i * F
                for f in range(F):
                    acc = acc + val_v[i, f] * rows_v[rb + f, :]
                out_v[i, :] = acc
                return 0

            lax.fori_loop(0, CB, body, 0)
            pltpu.sync_copy(out_v, out_hbm.at[pl.ds(base, CB)])

    return feature_linear


_kernel_fn = _make_kernel()


@jax.jit
def kernel(feature_idx, feature_value, weight, bias):
    idx2d = feature_idx.reshape(B * F // 128, 128)
    return _kernel_fn(idx2d, feature_value, weight, bias)


# speedup vs baseline: 1.0963x; 1.0963x over previous
import functools

import jax
import jax.numpy as jnp
from jax import lax
from jax.experimental import pallas as pl
from jax.experimental.pallas import tpu as pltpu
from jax.experimental.pallas import tpu_sc as plsc

B = 16384
F = 26
D = 16

_INFO = plsc.get_sparse_core_info()
NC = _INFO.num_cores
NS = _INFO.num_subcores
NW = NC * NS

CB = 64
ROWS = CB * F
G = ROWS // 128
B_PER_W = B // NW
NCHUNK = B_PER_W // CB


def _make_kernel():
    mesh = plsc.VectorSubcoreMesh(core_axis_name="c", subcore_axis_name="s")

    @functools.partial(
        pl.kernel,
        mesh=mesh,
        out_type=jax.ShapeDtypeStruct((B, D), jnp.float32),
        scratch_types=[
            pltpu.VMEM((G, 128), jnp.int32),
            pltpu.VMEM((CB, 32), jnp.float32),
            pltpu.VMEM((ROWS, D), jnp.float32),
            pltpu.VMEM((CB, D), jnp.float32),
            pltpu.VMEM((D,), jnp.float32),
            pltpu.SemaphoreType.DMA,
        ],
        compiler_params=pltpu.CompilerParams(use_tc_tiling_on_sc=False),
    )
    def feature_linear(idx_hbm, val_hbm, table_hbm, bias_hbm, out_hbm,
                       idx_v, val_v, rows_v, out_v, bias_v, sem):
        wid = lax.axis_index("s") * NC + lax.axis_index("c")
        pltpu.sync_copy(bias_hbm, bias_v)

        for c in range(NCHUNK):
            base = wid * B_PER_W + c * CB
            goff = base * F // 128
            pltpu.sync_copy(idx_hbm.at[pl.ds(goff, G)], idx_v)
            pltpu.sync_copy(val_hbm.at[pl.ds(base, CB)], val_v)

            copies = []
            for g in range(G):
                copies.append(
                    pltpu.async_copy(
                        table_hbm.at[idx_v.at[g]],
                        rows_v.at[pl.ds(g * 128, 128)],
                        sem,
                    )
                )
            for cp in copies:
                cp.wait()

            def body(i, _):
                acc = bias_v[:]
                rb = i * F
                vlo = val_v[i, 0:16]
                vhi = val_v[i, 16:32]
                for f in range(F):
                    v = vlo[f] if f < 16 else vhi[f - 16]
                    acc = acc + v * rows_v[rb + f, :]
                out_v[i, :] = acc
                return 0

            lax.fori_loop(0, CB, body, 0)
            pltpu.sync_copy(out_v, out_hbm.at[pl.ds(base, CB)])

    return feature_linear


_kernel_fn = _make_kernel()


@jax.jit
def kernel(feature_idx, feature_value, weight, bias):
    idx2d = feature_idx.reshape(B * F // 128, 128)
    val_pad = jnp.pad(feature_value, ((0, 0), (0, 32 - F)))
    return _kernel_fn(idx2d, val_pad, weight, bias)
